# baseline (device time: 18521 ns/iter reference)
import jax
import jax.numpy as jnp
from jax import lax
from jax.experimental import pallas as pl
from jax.experimental.pallas import tpu as pltpu

N_DEV = 8
R = 3


def kernel(x):
    m_per, n = x.shape
    M = N_DEV * m_per

    def packed_idx(F):
        row = lax.broadcasted_iota(jnp.int32, (F, 2 * n), 0)
        lane = lax.broadcasted_iota(jnp.int32, (F, 2 * n), 1)
        return row + jnp.where(lane >= n, F, 0)

    def cmpx(P, idxp, F, j, dirmask):
        up = (idxp & j) == 0
        if j == F:
            p = jnp.concatenate([P[:, n:], P[:, :n]], axis=1)
        else:
            p = jnp.where(up, pltpu.roll(P, F - j, 0), pltpu.roll(P, j, 0))
        take_min = up == dirmask
        return jnp.where(take_min, jnp.minimum(P, p), jnp.maximum(P, p))

    def body(x_ref, out_ref, gather_ref, send_sems, recv_sems):
        my = lax.axis_index("i")

        barrier_sem = pltpu.get_barrier_semaphore()
        for r in range(R):
            pl.semaphore_signal(
                barrier_sem, inc=1,
                device_id=(my ^ (1 << r),),
                device_id_type=pl.DeviceIdType.MESH,
            )
        pl.semaphore_wait(barrier_sem, R)

        def load_packed(start, F):
            return jnp.concatenate(
                [gather_ref[pl.ds(start, F), :],
                 gather_ref[pl.ds(start + F, F), :]],
                axis=1,
            )

        def store_packed(P, start, F):
            gather_ref[pl.ds(start, F), :] = P[:, :n]
            gather_ref[pl.ds(start + F, F), :] = P[:, n:]

        F = m_per // 2
        P = jnp.concatenate([x_ref[:F, :], x_ref[F:, :]], axis=1)
        idxp = packed_idx(F)
        flip = (my & 1) == 1
        k = 2
        while k <= m_per:
            j = k // 2
            while j >= 1:
                P = cmpx(P, idxp, F, j, ((idxp & k) == 0) ^ flip)
                j //= 2
            k *= 2
        store_packed(P, my * m_per, F)

        for r in range(R):
            s = 1 << r
            rows = s * m_per
            partner = my ^ s
            bstart = (my - (my & (s - 1))) * m_per
            rdma = pltpu.make_async_remote_copy(
                src_ref=gather_ref.at[pl.ds(bstart, rows)],
                dst_ref=gather_ref.at[pl.ds(bstart, rows)],
                send_sem=send_sems.at[r],
                recv_sem=recv_sems.at[r],
                device_id=(partner,),
                device_id_type=pl.DeviceIdType.MESH,
            )
            rdma.start()
            rdma.wait_recv()

            F = rows
            b2start = (my - (my & (2 * s - 1))) * m_per
            d_asc = ((my >> (r + 1)) & 1) == 0
            idxp = packed_idx(F)
            if r < R - 1:
                P = load_packed(b2start, F)
                j = rows
                while j >= 1:
                    P = cmpx(P, idxp, F, j, d_asc)
                    j //= 2
                rdma.wait_send()
                store_packed(P, b2start, F)
            else:
                P = load_packed(0, F)
                for j in (rows, rows // 2, rows // 4):
                    P = cmpx(P, idxp, F, j, True)
                rdma.wait_send()
                store_packed(P, 0, F)
                Fs = m_per // 2
                sl = load_packed(my * m_per, Fs)
                idxs = packed_idx(Fs)
                j = Fs
                while j >= 1:
                    sl = cmpx(sl, idxs, Fs, j, True)
                    j //= 2
                out_ref[:Fs, :] = sl[:, :n]
                out_ref[Fs:, :] = sl[:, n:]

    return pl.pallas_call(
        body,
        out_shape=jax.ShapeDtypeStruct((m_per, n), x.dtype),
        in_specs=[pl.BlockSpec(memory_space=pltpu.VMEM)],
        out_specs=pl.BlockSpec(memory_space=pltpu.VMEM),
        scratch_shapes=[
            pltpu.VMEM((M, n), x.dtype),
            pltpu.SemaphoreType.DMA((R,)),
            pltpu.SemaphoreType.DMA((R,)),
        ],
        compiler_params=pltpu.CompilerParams(collective_id=0),
    )(x)


# device time: 13071 ns/iter; 1.4170x vs baseline; 1.4170x over previous
import jax
import jax.numpy as jnp
from jax import lax
from jax.experimental import pallas as pl
from jax.experimental.pallas import tpu as pltpu

N_DEV = 8


def _cmpx(v, idx, j, dirmask):
    L = v.shape[0]
    up = (idx & j) == 0
    p = jnp.where(up, pltpu.roll(v, L - j, 0), pltpu.roll(v, j, 0))
    take_min = up == dirmask
    return jnp.where(take_min, jnp.minimum(v, p), jnp.maximum(v, p))


def kernel(x):
    m_per, n = x.shape
    M = N_DEV * m_per

    def body(x_ref, out_ref, gather_ref, merge_ref, send_sems, recv_sems):
        my = lax.axis_index("i")

        barrier_sem = pltpu.get_barrier_semaphore()
        for e in range(1, N_DEV):
            pl.semaphore_signal(
                barrier_sem, inc=1,
                device_id=(my ^ e,),
                device_id_type=pl.DeviceIdType.MESH,
            )
        pl.semaphore_wait(barrier_sem, N_DEV - 1)

        v = x_ref[:, :]
        flip = (my & 1) == 1
        idx = lax.broadcasted_iota(jnp.int32, (m_per, n), 0)
        k = 2
        while k <= m_per:
            j = k // 2
            while j >= 1:
                v = _cmpx(v, idx, j, ((idx & k) == 0) ^ flip)
                j //= 2
            k *= 2
        gather_ref[pl.ds(my * m_per, m_per), :] = v

        rdmas = {}
        for e in range(1, N_DEV):
            rdma = pltpu.make_async_remote_copy(
                src_ref=gather_ref.at[pl.ds(my * m_per, m_per)],
                dst_ref=gather_ref.at[pl.ds(my * m_per, m_per)],
                send_sem=send_sems.at[e - 1],
                recv_sem=recv_sems.at[e - 1],
                device_id=(my ^ e,),
                device_id_type=pl.DeviceIdType.MESH,
            )
            rdma.start()
            rdmas[e] = rdma

        def merge_pair(origin):
            start = (origin & ~1) * m_per
            w = gather_ref[pl.ds(start, 2 * m_per), :]
            widx = lax.broadcasted_iota(jnp.int32, (2 * m_per, n), 0)
            d_asc = ((origin >> 1) & 1) == 0
            j = m_per
            while j >= 1:
                w = _cmpx(w, widx, j, d_asc)
                j //= 2
            merge_ref[pl.ds(start, 2 * m_per), :] = w

        def merge_512(origin):
            start = (origin & ~3) * m_per
            w = merge_ref[pl.ds(start, 4 * m_per), :]
            widx = lax.broadcasted_iota(jnp.int32, (4 * m_per, n), 0)
            d_asc = ((origin >> 2) & 1) == 0
            j = 2 * m_per
            while j >= 1:
                w = _cmpx(w, widx, j, d_asc)
                j //= 2
            merge_ref[pl.ds(start, 4 * m_per), :] = w

        rdmas[1].wait_recv()
        merge_pair(my)
        rdmas[2].wait_recv()
        rdmas[3].wait_recv()
        merge_pair(my ^ 2)
        merge_512(my)

        rdmas[4].wait_recv()
        rdmas[5].wait_recv()
        merge_pair(my ^ 4)
        rdmas[6].wait_recv()
        rdmas[7].wait_recv()
        merge_pair(my ^ 6)
        merge_512(my ^ 4)

        w = merge_ref[:, :]
        widx = lax.broadcasted_iota(jnp.int32, (M, n), 0)
        for j in (M // 2, M // 4, M // 8):
            w = _cmpx(w, widx, j, True)
        merge_ref[:, :] = w

        sl = merge_ref[pl.ds(my * m_per, m_per), :]
        j = m_per // 2
        while j >= 1:
            sl = _cmpx(sl, idx, j, True)
            j //= 2
        out_ref[:, :] = sl

        for e in range(1, N_DEV):
            rdmas[e].wait_send()

    return pl.pallas_call(
        body,
        out_shape=jax.ShapeDtypeStruct((m_per, n), x.dtype),
        in_specs=[pl.BlockSpec(memory_space=pltpu.VMEM)],
        out_specs=pl.BlockSpec(memory_space=pltpu.VMEM),
        scratch_shapes=[
            pltpu.VMEM((M, n), x.dtype),
            pltpu.VMEM((M, n), x.dtype),
            pltpu.SemaphoreType.DMA((N_DEV - 1,)),
            pltpu.SemaphoreType.DMA((N_DEV - 1,)),
        ],
        compiler_params=pltpu.CompilerParams(collective_id=0),
    )(x)


# device time: 13053 ns/iter; 1.4189x vs baseline; 1.0014x over previous
import jax
import jax.numpy as jnp
from jax import lax
from jax.experimental import pallas as pl
from jax.experimental.pallas import tpu as pltpu

N_DEV = 8


def _cmpx(v, idx, j, dirmask):
    L = v.shape[0]
    up = (idx & j) == 0
    p = jnp.where(up, pltpu.roll(v, L - j, 0), pltpu.roll(v, j, 0))
    take_min = up == dirmask
    return jnp.where(take_min, jnp.minimum(v, p), jnp.maximum(v, p))


def kernel(x):
    m_per, n = x.shape
    M = N_DEV * m_per

    def body(x_ref, out_ref, gather_ref, merge_ref, send_sems, recv_sems):
        my = lax.axis_index("i")

        barrier_sem = pltpu.get_barrier_semaphore()
        for e in range(1, N_DEV):
            pl.semaphore_signal(
                barrier_sem, inc=1,
                device_id=(my ^ e,),
                device_id_type=pl.DeviceIdType.MESH,
            )
        pl.semaphore_wait(barrier_sem, N_DEV - 1)

        v = x_ref[:, :]
        flip = (my & 1) == 1
        idx = lax.broadcasted_iota(jnp.int32, (m_per, n), 0)
        k = 2
        while k <= m_per:
            j = k // 2
            while j >= 1:
                v = _cmpx(v, idx, j, ((idx & k) == 0) ^ flip)
                j //= 2
            k *= 2
        gather_ref[pl.ds(my * m_per, m_per), :] = v

        rdmas = {}
        for e in range(1, N_DEV):
            rdma = pltpu.make_async_remote_copy(
                src_ref=gather_ref.at[pl.ds(my * m_per, m_per)],
                dst_ref=gather_ref.at[pl.ds(my * m_per, m_per)],
                send_sem=send_sems.at[e - 1],
                recv_sem=recv_sems.at[e - 1],
                device_id=(my ^ e,),
                device_id_type=pl.DeviceIdType.MESH,
            )
            rdma.start()
            rdmas[e] = rdma

        def merge_pair(origin):
            start = (origin & ~1) * m_per
            w = gather_ref[pl.ds(start, 2 * m_per), :]
            widx = lax.broadcasted_iota(jnp.int32, (2 * m_per, n), 0)
            d_asc = ((origin >> 1) & 1) == 0
            j = m_per
            while j >= 1:
                w = _cmpx(w, widx, j, d_asc)
                j //= 2
            merge_ref[pl.ds(start, 2 * m_per), :] = w

        def merge_512(origin):
            start = (origin & ~3) * m_per
            w = merge_ref[pl.ds(start, 4 * m_per), :]
            widx = lax.broadcasted_iota(jnp.int32, (4 * m_per, n), 0)
            d_asc = ((origin >> 2) & 1) == 0
            j = 2 * m_per
            while j >= 1:
                w = _cmpx(w, widx, j, d_asc)
                j //= 2
            merge_ref[pl.ds(start, 4 * m_per), :] = w

        rdmas[1].wait_recv()
        merge_pair(my)
        rdmas[2].wait_recv()
        rdmas[3].wait_recv()
        merge_pair(my ^ 2)
        merge_512(my)

        rdmas[4].wait_recv()
        rdmas[5].wait_recv()
        merge_pair(my ^ 4)
        rdmas[6].wait_recv()
        rdmas[7].wait_recv()
        merge_pair(my ^ 6)
        merge_512(my ^ 4)

        half = (my >> 2) & 1
        mine = merge_ref[pl.ds(half * (M // 2), M // 2), :]
        other = merge_ref[pl.ds((1 - half) * (M // 2), M // 2), :]
        w = jnp.where(half == 0, jnp.minimum(mine, other),
                      jnp.maximum(mine, other))
        merge_ref[pl.ds(half * (M // 2), M // 2), :] = w

        q = my >> 1
        mine = merge_ref[pl.ds(q * (M // 4), M // 4), :]
        other = merge_ref[pl.ds((q ^ 1) * (M // 4), M // 4), :]
        w = jnp.where((q & 1) == 0, jnp.minimum(mine, other),
                      jnp.maximum(mine, other))
        merge_ref[pl.ds(q * (M // 4), M // 4), :] = w

        mine = merge_ref[pl.ds(my * m_per, m_per), :]
        other = merge_ref[pl.ds((my ^ 1) * m_per, m_per), :]
        sl = jnp.where((my & 1) == 0, jnp.minimum(mine, other),
                       jnp.maximum(mine, other))

        j = m_per // 2
        while j >= 1:
            sl = _cmpx(sl, idx, j, True)
            j //= 2
        out_ref[:, :] = sl

        for e in range(1, N_DEV):
            rdmas[e].wait_send()

    return pl.pallas_call(
        body,
        out_shape=jax.ShapeDtypeStruct((m_per, n), x.dtype),
        in_specs=[pl.BlockSpec(memory_space=pltpu.VMEM)],
        out_specs=pl.BlockSpec(memory_space=pltpu.VMEM),
        scratch_shapes=[
            pltpu.VMEM((M, n), x.dtype),
            pltpu.VMEM((M, n), x.dtype),
            pltpu.SemaphoreType.DMA((N_DEV - 1,)),
            pltpu.SemaphoreType.DMA((N_DEV - 1,)),
        ],
        compiler_params=pltpu.CompilerParams(collective_id=0),
    )(x)
